# baseline (speedup 1.0000x reference)
import functools

import jax
import jax.numpy as jnp
import numpy as np
from jax.experimental import pallas as pl
from jax.experimental.pallas import tpu as pltpu

B, S, DIN = 2, 2048, 128
D, EDIM, L, H = 256, 128, 4, 4
DH = D // H
N = B * S
OFFS = (-3, -2, -1, 1, 2, 3)


def _layernorm(x, s, b, eps=1e-5):
    m = jnp.mean(x, axis=-1, keepdims=True)
    v = jnp.mean((x - m) ** 2, axis=-1, keepdims=True)
    return (x - m) / jnp.sqrt(v + eps) * s + b


def _fwd(x_ref, oh_ref, te_ref, Win_ref, bin_ref, erow_ref,
         Wq_ref, Wk_ref, Wv_ref, We_ref, Wo_ref, lns_ref, lnb_ref,
         Wg_ref, bg_ref, Wc_ref, bc_ref, lnos_ref, lnob_ref,
         Wout_ref, bout_ref, out_ref):
    f32 = jnp.float32
    dot = functools.partial(jnp.dot, preferred_element_type=f32)

    def bdot(a, b):
        return jnp.dot(a.astype(jnp.bfloat16), b.astype(jnp.bfloat16),
                       preferred_element_type=f32)

    h = bdot(x_ref[...], Win_ref[...]) + bin_ref[...]
    h = h + dot(oh_ref[...], te_ref[...])
    h16 = h.astype(jnp.bfloat16)

    bf16 = jnp.bfloat16

    pos = jax.lax.broadcasted_iota(jnp.int32, (S, 1), 0)
    maskb = [jnp.where((pos + o >= 0) & (pos + o < S), f32(0), f32(-1e9))
             for o in OFFS]

    di = jax.lax.broadcasted_iota(jnp.int32, (D, H), 0)
    hi = jax.lax.broadcasted_iota(jnp.int32, (D, H), 1)
    seg = (di // DH == hi).astype(f32)
    segT = seg.T
    seg16 = (seg * f32(1.0 / np.sqrt(DH))).astype(bf16)
    mean4 = jnp.full((D, H), 1.0 / D, f32)

    def ln(xv, sc, bi, eps=1e-5):
        m4 = dot(xv, mean4)
        v4 = dot(xv * xv, mean4) - m4 * m4
        r4 = 1.0 / jnp.sqrt(v4 + eps)
        a = dot(r4, segT)
        bm = dot(m4 * r4, segT)
        return (xv * a - bm) * sc + bi

    for l in range(L):
        q = jnp.dot(h16, Wq_ref[l].astype(bf16),
                    preferred_element_type=f32).astype(bf16)
        k = jnp.dot(h16, Wk_ref[l].astype(bf16),
                    preferred_element_type=f32).astype(bf16)
        v = jnp.dot(h16, Wv_ref[l].astype(bf16),
                    preferred_element_type=f32).astype(bf16)
        e = dot(erow_ref[...], We_ref[l])
        kv = jnp.concatenate([k + e.astype(bf16), v], axis=1)
        shifted = [jnp.roll(kv, -o, axis=0) for o in OFFS]

        exs = [jnp.exp(dot(q * kvs[:, :D], seg16) + mb)
               for kvs, mb in zip(shifted, maskb)]
        den = exs[0]
        for ex in exs[1:]:
            den = den + ex

        agg = None
        for kvs, ex in zip(shifted, exs):
            t = dot(ex, segT) * kvs[:, D:]
            agg = t if agg is None else agg + t
        denx = dot(den, segT)
        agg = agg / (denx + 1e-9) + e

        h = ln(h + bdot(agg, Wo_ref[l]),
               lns_ref[l:l + 1], lnb_ref[l:l + 1])
        h16 = h.astype(bf16)

    gate = jax.nn.sigmoid(bdot(h, Wg_ref[...]) + bg_ref[...])
    c = jnp.tanh(bdot(h, Wc_ref[...]) + bc_ref[...])
    h = gate * h + (1.0 - gate) * c
    h = ln(h, lnos_ref[...], lnob_ref[...])
    out_ref[...] = bdot(h, Wout_ref[...]) + bout_ref[...]


@jax.jit
def kernel(x, token_types, type_emb, W_in, b_in, edge_emb, Wq, Wk, Wv, We, Wo,
           ln_s, ln_b, Wg, bg, Wc, bc, lno_s, lno_b, W_out, b_out):
    x2 = x.reshape(N, DIN)
    oh = jax.nn.one_hot(token_types.reshape(-1), 8, dtype=jnp.float32)
    te = jnp.concatenate([type_emb, jnp.zeros((2, D), jnp.float32)], axis=0)
    erow = edge_emb[5:6]

    def full(a):
        return pl.BlockSpec(a.shape, lambda i: tuple(0 for _ in a.shape))

    weights = (W_in, b_in.reshape(1, D), erow,
               Wq, Wk, Wv, We, Wo, ln_s, ln_b,
               Wg, bg.reshape(1, D), Wc, bc.reshape(1, D),
               lno_s.reshape(1, D), lno_b.reshape(1, D),
               W_out, b_out.reshape(1, DIN))

    out = pl.pallas_call(
        _fwd,
        grid=(B,),
        in_specs=[pl.BlockSpec((S, DIN), lambda i: (i, 0)),
                  pl.BlockSpec((S, 8), lambda i: (i, 0)),
                  full(te)] + [full(w) for w in weights],
        out_specs=pl.BlockSpec((S, DIN), lambda i: (i, 0)),
        out_shape=jax.ShapeDtypeStruct((N, DIN), jnp.float32),
        compiler_params=pltpu.CompilerParams(
            dimension_semantics=("parallel",),
            vmem_limit_bytes=120 * 1024 * 1024),
    )(x2, oh, te, *weights)
    return out.reshape(B, S, DIN)


# speedup vs baseline: 1.0288x; 1.0288x over previous
import functools

import jax
import jax.numpy as jnp
import numpy as np
from jax.experimental import pallas as pl
from jax.experimental.pallas import tpu as pltpu

B, S, DIN = 2, 2048, 128
D, EDIM, L, H = 256, 128, 4, 4
DH = D // H
N = B * S
OFFS = (-3, -2, -1, 1, 2, 3)


def _layernorm(x, s, b, eps=1e-5):
    m = jnp.mean(x, axis=-1, keepdims=True)
    v = jnp.mean((x - m) ** 2, axis=-1, keepdims=True)
    return (x - m) / jnp.sqrt(v + eps) * s + b


def _fwd(x_ref, oh_ref, te_ref, Win_ref, bin_ref, erow_ref,
         Wq_ref, Wk_ref, Wv_ref, We_ref, Wo_ref, lns_ref, lnb_ref,
         Wg_ref, bg_ref, Wc_ref, bc_ref, lnos_ref, lnob_ref,
         Wout_ref, bout_ref, out_ref):
    f32 = jnp.float32
    dot = functools.partial(jnp.dot, preferred_element_type=f32)

    def bdot(a, b):
        return jnp.dot(a.astype(jnp.bfloat16), b.astype(jnp.bfloat16),
                       preferred_element_type=f32)

    h = bdot(x_ref[...], Win_ref[...]) + bin_ref[...]
    h = h + dot(oh_ref[...], te_ref[...])
    h16 = h.astype(jnp.bfloat16)

    bf16 = jnp.bfloat16

    pos = jax.lax.broadcasted_iota(jnp.int32, (S, 1), 0)
    maskb = [jnp.where((pos + o >= 0) & (pos + o < S), f32(0), f32(-1e9))
             for o in OFFS]

    di = jax.lax.broadcasted_iota(jnp.int32, (D, H), 0)
    hi = jax.lax.broadcasted_iota(jnp.int32, (D, H), 1)
    seg = (di // DH == hi).astype(f32)
    segT = seg.T
    seg16 = (seg * f32(1.0 / np.sqrt(DH))).astype(bf16)
    ln = _layernorm

    for l in range(L):
        q = jnp.dot(h16, Wq_ref[l].astype(bf16),
                    preferred_element_type=f32).astype(bf16)
        k = jnp.dot(h16, Wk_ref[l].astype(bf16),
                    preferred_element_type=f32).astype(bf16)
        v = jnp.dot(h16, Wv_ref[l].astype(bf16),
                    preferred_element_type=f32).astype(bf16)
        e = dot(erow_ref[...], We_ref[l])
        kv = jnp.concatenate([k + e.astype(bf16), v], axis=1)
        shifted = [jnp.roll(kv, -o, axis=0) for o in OFFS]

        exs = [jnp.exp(dot(q * kvs[:, :D], seg16) + mb)
               for kvs, mb in zip(shifted, maskb)]
        den = exs[0]
        for ex in exs[1:]:
            den = den + ex

        agg = None
        for kvs, ex in zip(shifted, exs):
            t = dot(ex, segT) * kvs[:, D:]
            agg = t if agg is None else agg + t
        denx = dot(den, segT)
        agg = agg / (denx + 1e-9) + e

        h = ln(h + bdot(agg, Wo_ref[l]),
               lns_ref[l:l + 1], lnb_ref[l:l + 1])
        h16 = h.astype(bf16)

    gate = jax.nn.sigmoid(bdot(h, Wg_ref[...]) + bg_ref[...])
    c = jnp.tanh(bdot(h, Wc_ref[...]) + bc_ref[...])
    h = gate * h + (1.0 - gate) * c
    h = ln(h, lnos_ref[...], lnob_ref[...])
    out_ref[...] = bdot(h, Wout_ref[...]) + bout_ref[...]


@jax.jit
def kernel(x, token_types, type_emb, W_in, b_in, edge_emb, Wq, Wk, Wv, We, Wo,
           ln_s, ln_b, Wg, bg, Wc, bc, lno_s, lno_b, W_out, b_out):
    x2 = x.reshape(N, DIN)
    oh = jax.nn.one_hot(token_types.reshape(-1), 8, dtype=jnp.float32)
    te = jnp.concatenate([type_emb, jnp.zeros((2, D), jnp.float32)], axis=0)
    erow = edge_emb[5:6]

    def full(a):
        return pl.BlockSpec(a.shape, lambda i: tuple(0 for _ in a.shape))

    weights = (W_in, b_in.reshape(1, D), erow,
               Wq, Wk, Wv, We, Wo, ln_s, ln_b,
               Wg, bg.reshape(1, D), Wc, bc.reshape(1, D),
               lno_s.reshape(1, D), lno_b.reshape(1, D),
               W_out, b_out.reshape(1, DIN))

    out = pl.pallas_call(
        _fwd,
        grid=(B,),
        in_specs=[pl.BlockSpec((S, DIN), lambda i: (i, 0)),
                  pl.BlockSpec((S, 8), lambda i: (i, 0)),
                  full(te)] + [full(w) for w in weights],
        out_specs=pl.BlockSpec((S, DIN), lambda i: (i, 0)),
        out_shape=jax.ShapeDtypeStruct((N, DIN), jnp.float32),
        compiler_params=pltpu.CompilerParams(
            dimension_semantics=("parallel",),
            vmem_limit_bytes=120 * 1024 * 1024),
    )(x2, oh, te, *weights)
    return out.reshape(B, S, DIN)


# speedup vs baseline: 1.0869x; 1.0565x over previous
import functools

import jax
import jax.numpy as jnp
import numpy as np
from jax.experimental import pallas as pl
from jax.experimental.pallas import tpu as pltpu

B, S, DIN = 2, 2048, 128
D, EDIM, L, H = 256, 128, 4, 4
DH = D // H
N = B * S
OFFS = (-3, -2, -1, 1, 2, 3)


def _layernorm(x, s, b, eps=1e-5):
    m = jnp.mean(x, axis=-1, keepdims=True)
    v = jnp.mean((x - m) ** 2, axis=-1, keepdims=True)
    return (x - m) / jnp.sqrt(v + eps) * s + b


def _fwd(x_ref, oh_ref, te_ref, Win_ref, bin_ref, erow_ref,
         Wq_ref, Wk_ref, Wv_ref, We_ref, Wo_ref, lns_ref, lnb_ref,
         Wg_ref, bg_ref, Wc_ref, bc_ref, lnos_ref, lnob_ref,
         Wout_ref, bout_ref, out_ref):
    f32 = jnp.float32
    bf16 = jnp.bfloat16
    dot = functools.partial(jnp.dot, preferred_element_type=f32)

    def bdot(a, b):
        return jnp.dot(a.astype(bf16), b.astype(bf16),
                       preferred_element_type=f32)

    h = bdot(x_ref[...], Win_ref[...]) + bin_ref[...]
    h = h + dot(oh_ref[...], te_ref[...])
    h16 = h.astype(bf16)

    pos = jax.lax.broadcasted_iota(jnp.int32, (S, 1), 0)
    maskb = [jnp.where((pos + o >= 0) & (pos + o < S), f32(0), f32(-1e9))
             for o in OFFS]

    d2 = jax.lax.broadcasted_iota(jnp.int32, (2 * D, 2 * H), 0)
    c2 = jax.lax.broadcasted_iota(jnp.int32, (2 * D, 2 * H), 1)
    match = (d2 // D == c2 // H) & ((d2 % D) // DH == c2 % H)
    seg2 = (match.astype(f32) * f32(1.0 / np.sqrt(DH))).astype(bf16)
    seg2T = jnp.transpose(match.astype(f32))

    for l in range(L):
        q4 = bdot(h16, Wq_ref[l]).astype(bf16)
        k4 = bdot(h16, Wk_ref[l])
        v4 = bdot(h16, Wv_ref[l]).astype(bf16)
        e = dot(erow_ref[...], We_ref[l])
        ke4 = (k4 + e).astype(bf16)
        qp = jnp.concatenate([q4[:S], q4[S:]], axis=1)
        kvp = jnp.concatenate([ke4[:S], ke4[S:], v4[:S], v4[S:]], axis=1)
        shifted = [jnp.roll(kvp, -o, axis=0) for o in OFFS]

        exs = [jnp.exp(dot(qp * kvs[:, :2 * D], seg2) + mb)
               for kvs, mb in zip(shifted, maskb)]
        den = exs[0]
        for ex in exs[1:]:
            den = den + ex

        agg = None
        for kvs, ex in zip(shifted, exs):
            t = dot(ex, seg2T) * kvs[:, 2 * D:]
            agg = t if agg is None else agg + t
        denx = dot(den, seg2T)
        agg = agg / (denx + 1e-9)
        agg4 = jnp.concatenate([agg[:, :D], agg[:, D:]], axis=0) + e

        h = _layernorm(h + bdot(agg4, Wo_ref[l]),
                       lns_ref[l:l + 1], lnb_ref[l:l + 1])
        h16 = h.astype(bf16)

    gate = jax.nn.sigmoid(bdot(h, Wg_ref[...]) + bg_ref[...])
    c = jnp.tanh(bdot(h, Wc_ref[...]) + bc_ref[...])
    h = gate * h + (1.0 - gate) * c
    h = _layernorm(h, lnos_ref[...], lnob_ref[...])
    out_ref[...] = bdot(h, Wout_ref[...]) + bout_ref[...]


@jax.jit
def kernel(x, token_types, type_emb, W_in, b_in, edge_emb, Wq, Wk, Wv, We, Wo,
           ln_s, ln_b, Wg, bg, Wc, bc, lno_s, lno_b, W_out, b_out):
    x2 = x.reshape(N, DIN)
    oh = jax.nn.one_hot(token_types.reshape(-1), 8, dtype=jnp.float32)
    te = jnp.concatenate([type_emb, jnp.zeros((2, D), jnp.float32)], axis=0)
    erow = edge_emb[5:6]

    out = pl.pallas_call(
        _fwd,
        out_shape=jax.ShapeDtypeStruct((N, DIN), jnp.float32),
        compiler_params=pltpu.CompilerParams(
            vmem_limit_bytes=120 * 1024 * 1024),
    )(x2, oh, te, W_in, b_in.reshape(1, D), erow,
      Wq, Wk, Wv, We, Wo, ln_s, ln_b,
      Wg, bg.reshape(1, D), Wc, bc.reshape(1, D),
      lno_s.reshape(1, D), lno_b.reshape(1, D),
      W_out, b_out.reshape(1, DIN))
    return out.reshape(B, S, DIN)
